# baseline (speedup 1.0000x reference)
import functools

import jax
import jax.numpy as jnp
from jax import lax
from jax.experimental import pallas as pl
from jax.experimental.pallas import tpu as pltpu
from jax.experimental.pallas import tpu_sc as plsc

_COMMITMENT_COST = 0.25
_K = 8192
_D = 256
_TOK_BLOCK = 256


_LANE = 128


def _distance_body(z_ref, z2_ref, e2_ref, lane_ref, e_ref, idx_ref,
                   dminsum_ref):
    emb = e_ref[...]
    e2 = e2_ref[...]
    lane = lane_ref[...]
    nblk = z_ref.shape[0] // _TOK_BLOCK
    nchunk = _K // _LANE
    total = jnp.zeros((1, 1), jnp.float32)
    for b in range(nblk):
        zb = z_ref[b * _TOK_BLOCK:(b + 1) * _TOK_BLOCK, :]
        z2 = z2_ref[b * _TOK_BLOCK:(b + 1) * _TOK_BLOCK, :]
        s2 = lax.dot_general(zb, emb, (((1,), (1,)), ((), ())),
                             preferred_element_type=jnp.float32)
        z2b = jnp.broadcast_to(z2, (_TOK_BLOCK, _LANE))
        m = (z2b + lax.slice(e2, (0, 0), (1, _LANE))) \
            - lax.slice(s2, (0, 0), (_TOK_BLOCK, _LANE))
        a = jnp.zeros((_TOK_BLOCK, _LANE), jnp.float32)
        for j in range(1, nchunk):
            ej = lax.slice(e2, (0, j * _LANE), (1, (j + 1) * _LANE))
            sj = lax.slice(s2, (0, j * _LANE), (_TOK_BLOCK, (j + 1) * _LANE))
            dj = (z2b + ej) - sj
            lt = dj < m
            m = jnp.minimum(m, dj)
            a = jnp.where(lt, jnp.float32(j * _LANE), a)
        dmin = jnp.min(m, axis=1, keepdims=True)
        key = a + lane
        hit = jnp.where(m == dmin, key, jnp.float32(_K))
        idx_ref[b, 0, :] = jnp.min(hit, axis=1).astype(jnp.int32)
        total = total + jnp.sum(dmin, axis=0, keepdims=True)
    dminsum_ref[...] = total


def _distances(z_flat, z2, e2, iota_f, embedding2):
    n_tok = z_flat.shape[0]
    nblk = n_tok // _TOK_BLOCK
    return pl.pallas_call(
        _distance_body,
        grid=(1,),
        in_specs=[
            pl.BlockSpec((n_tok, _D), lambda i: (0, 0)),
            pl.BlockSpec((n_tok, 1), lambda i: (0, 0)),
            pl.BlockSpec((1, _K), lambda i: (0, 0)),
            pl.BlockSpec((1, _LANE), lambda i: (0, 0)),
            pl.BlockSpec((_K, _D), lambda i: (0, 0)),
        ],
        out_specs=[
            pl.BlockSpec((nblk, 1, _TOK_BLOCK), lambda i: (0, 0, 0)),
            pl.BlockSpec((1, 1), lambda i: (0, 0)),
        ],
        out_shape=[
            jax.ShapeDtypeStruct((nblk, 1, _TOK_BLOCK), jnp.int32),
            jax.ShapeDtypeStruct((1, 1), jnp.float32),
        ],
    )(z_flat, z2, e2, iota_f, embedding2)


def _gather_rows(embedding, idx):
    n_tok = idx.shape[0]
    info = plsc.get_sparse_core_info()
    nc, ns = info.num_cores, info.num_subcores
    nw = nc * ns
    b_per_w = n_tok // nw
    mesh = plsc.VectorSubcoreMesh(core_axis_name="c", subcore_axis_name="s")

    @functools.partial(
        pl.kernel, mesh=mesh,
        out_type=jax.ShapeDtypeStruct((n_tok, _D), jnp.float32),
        scratch_types=[
            pltpu.VMEM((b_per_w,), jnp.int32),
            pltpu.VMEM((b_per_w, _D), jnp.float32),
            pltpu.SemaphoreType.DMA,
        ],
    )
    def gather_k(table_hbm, idx_hbm, out_hbm, idx_v, rows_v, sem):
        wid = lax.axis_index("s") * nc + lax.axis_index("c")
        base = wid * b_per_w
        pltpu.sync_copy(idx_hbm.at[pl.ds(base, b_per_w)], idx_v)
        pltpu.async_copy(table_hbm.at[idx_v], rows_v, sem).wait()
        pltpu.sync_copy(rows_v, out_hbm.at[pl.ds(base, b_per_w)])

    return gather_k(embedding, idx)


def kernel(z, embedding):
    z = z.astype(jnp.float32)
    b, c, h, w = z.shape
    z_flat = jnp.transpose(z, (0, 2, 3, 1)).reshape(-1, c)
    z2 = jnp.sum(z_flat ** 2, axis=1, keepdims=True)
    e2 = jnp.sum(embedding ** 2, axis=1)[None, :]
    lane_f = jnp.arange(_LANE, dtype=jnp.float32)[None, :]
    emb2 = embedding * 2.0
    n_tok = z_flat.shape[0]
    half = n_tok // 2
    bh = b // 2
    outs = []
    idxs = []
    dsum = jnp.zeros((), jnp.float32)
    for s in range(2):
        zs = lax.slice(z_flat, (s * half, 0), ((s + 1) * half, _D))
        z2s = lax.slice(z2, (s * half, 0), ((s + 1) * half, 1))
        idx3, dmin_sum = _distances(zs, z2s, e2, lane_f, emb2)
        idx = idx3.reshape(-1)
        zq_half = _gather_rows(embedding, idx).reshape(bh, h, w, c)
        outs.append(jnp.transpose(zq_half, (0, 3, 1, 2)))
        idxs.append(idx.reshape(bh, h, w))
        dsum = dsum + dmin_sum[0, 0]
    mean_sq = dsum / z.size
    commitment_loss = _COMMITMENT_COST * mean_sq
    codebook_loss = mean_sq
    loss = commitment_loss + codebook_loss
    z_quantized_out = jnp.concatenate(outs, axis=0)
    min_idx = jnp.concatenate(idxs, axis=0)
    return (z_quantized_out, loss, commitment_loss, codebook_loss, min_idx)


# speedup vs baseline: 1.1838x; 1.1838x over previous
import functools

import jax
import jax.numpy as jnp
from jax import lax
from jax.experimental import pallas as pl
from jax.experimental.pallas import tpu as pltpu
from jax.experimental.pallas import tpu_sc as plsc

_COMMITMENT_COST = 0.25
_K = 8192
_D = 256
_TOK_BLOCK = 256


_LANE = 128


def _distance_body(z_ref, z2_ref, e2_ref, lane_ref, e_ref, idx_ref,
                   dminsum_ref):
    emb = e_ref[...]
    e2 = e2_ref[...]
    lane = lane_ref[...]
    nblk = z_ref.shape[0] // _TOK_BLOCK
    nchunk = _K // _LANE
    total = jnp.zeros((1, 1), jnp.float32)
    for b in range(nblk):
        zb = z_ref[b * _TOK_BLOCK:(b + 1) * _TOK_BLOCK, :]
        z2 = z2_ref[b * _TOK_BLOCK:(b + 1) * _TOK_BLOCK, :]
        s2 = lax.dot_general(zb, emb, (((1,), (1,)), ((), ())),
                             preferred_element_type=jnp.float32)
        z2b = jnp.broadcast_to(z2, (_TOK_BLOCK, _LANE))
        m = (z2b + lax.slice(e2, (0, 0), (1, _LANE))) \
            - lax.slice(s2, (0, 0), (_TOK_BLOCK, _LANE))
        a = jnp.zeros((_TOK_BLOCK, _LANE), jnp.float32)
        for j in range(1, nchunk):
            ej = lax.slice(e2, (0, j * _LANE), (1, (j + 1) * _LANE))
            sj = lax.slice(s2, (0, j * _LANE), (_TOK_BLOCK, (j + 1) * _LANE))
            dj = (z2b + ej) - sj
            lt = dj < m
            m = jnp.minimum(m, dj)
            a = jnp.where(lt, jnp.float32(j * _LANE), a)
        dmin = jnp.min(m, axis=1, keepdims=True)
        key = a + lane
        hit = jnp.where(m == dmin, key, jnp.float32(_K))
        idx_ref[b, 0, :] = jnp.min(hit, axis=1).astype(jnp.int32)
        total = total + jnp.sum(dmin, axis=0, keepdims=True)
    dminsum_ref[...] = total


def _distances(z_flat, z2, e2, iota_f, embedding2):
    n_tok = z_flat.shape[0]
    nblk = n_tok // _TOK_BLOCK
    return pl.pallas_call(
        _distance_body,
        grid=(1,),
        in_specs=[
            pl.BlockSpec((n_tok, _D), lambda i: (0, 0)),
            pl.BlockSpec((n_tok, 1), lambda i: (0, 0)),
            pl.BlockSpec((1, _K), lambda i: (0, 0)),
            pl.BlockSpec((1, _LANE), lambda i: (0, 0)),
            pl.BlockSpec((_K, _D), lambda i: (0, 0)),
        ],
        out_specs=[
            pl.BlockSpec((nblk, 1, _TOK_BLOCK), lambda i: (0, 0, 0)),
            pl.BlockSpec((1, 1), lambda i: (0, 0)),
        ],
        out_shape=[
            jax.ShapeDtypeStruct((nblk, 1, _TOK_BLOCK), jnp.int32),
            jax.ShapeDtypeStruct((1, 1), jnp.float32),
        ],
    )(z_flat, z2, e2, iota_f, embedding2)


def _gather_rows(embedding, idx):
    n_tok = idx.shape[0]
    info = plsc.get_sparse_core_info()
    nc, ns = info.num_cores, info.num_subcores
    nw = nc * ns
    b_per_w = n_tok // nw
    mesh = plsc.VectorSubcoreMesh(core_axis_name="c", subcore_axis_name="s")

    @functools.partial(
        pl.kernel, mesh=mesh,
        out_type=jax.ShapeDtypeStruct((n_tok, _D), jnp.float32),
        scratch_types=[
            pltpu.VMEM((b_per_w,), jnp.int32),
            pltpu.VMEM((b_per_w, _D), jnp.float32),
            pltpu.SemaphoreType.DMA,
        ],
    )
    def gather_k(table_hbm, idx_hbm, out_hbm, idx_v, rows_v, sem):
        wid = lax.axis_index("s") * nc + lax.axis_index("c")
        base = wid * b_per_w
        pltpu.sync_copy(idx_hbm.at[pl.ds(base, b_per_w)], idx_v)
        pltpu.async_copy(table_hbm.at[idx_v], rows_v, sem).wait()
        pltpu.sync_copy(rows_v, out_hbm.at[pl.ds(base, b_per_w)])

    return gather_k(embedding, idx)


def kernel(z, embedding):
    z = z.astype(jnp.float32)
    b, c, h, w = z.shape
    z_flat = jnp.transpose(z, (0, 2, 3, 1)).reshape(-1, c)
    z2 = jnp.sum(z_flat ** 2, axis=1, keepdims=True)
    e2 = jnp.sum(embedding ** 2, axis=1)
    lane_f = jnp.arange(_LANE, dtype=jnp.float32)[None, :]
    idx3, dmin_sum = _distances(z_flat, z2, e2[None, :], lane_f,
                                embedding * 2.0)
    idx = idx3.reshape(-1)
    zq_flat = _gather_rows(embedding, idx)
    zq_bhwc = zq_flat.reshape(b, h, w, c)
    mean_sq = dmin_sum[0, 0] / z.size
    commitment_loss = _COMMITMENT_COST * mean_sq
    codebook_loss = mean_sq
    loss = commitment_loss + codebook_loss
    z_quantized_out = jnp.transpose(zq_bhwc, (0, 3, 1, 2))
    min_idx = idx.reshape(b, h, w)
    return (z_quantized_out, loss, commitment_loss, codebook_loss, min_idx)
